# baseline (speedup 1.0000x reference)
import functools

import jax
import jax.numpy as jnp
from jax.experimental import pallas as pl
from jax.experimental.pallas import tpu as pltpu
from jax.experimental.pallas import tpu_sc as plsc

_B, _N, _K = 4, 512, 8
_HID = 64
_NW = 32
_RPT = (_B * _N) // _NW
_NG = _RPT // 16
_TPG = _N // _RPT


@functools.partial(
    pl.kernel,
    out_type=jax.ShapeDtypeStruct((_B * _N * _N,), jnp.float32),
    mesh=plsc.VectorSubcoreMesh(core_axis_name="c", subcore_axis_name="s"),
    compiler_params=pltpu.CompilerParams(needs_layout_passes=False),
    scratch_types=[
        pltpu.VMEM((_N,), jnp.float32),
        pltpu.VMEM((_N,), jnp.float32),
        pltpu.VMEM((_NG * _N * 16,), jnp.float32),
        pltpu.VMEM((_RPT * _N,), jnp.float32),
    ],
)
def _knn_sc(ptst_hbm, s_hbm, pxv, pyv, d2buf, sbuf):
    f32 = jnp.float32
    lane = jax.lax.broadcasted_iota(jnp.int32, (16,), 0)
    wid = jax.lax.axis_index("s") * 2 + jax.lax.axis_index("c")
    b = wid // _TPG
    ib = (wid % _TPG) * _RPT

    pltpu.sync_copy(ptst_hbm.at[b, 0], pxv)
    pltpu.sync_copy(ptst_hbm.at[b, 1], pyv)

    zeros = jnp.zeros((16,), f32)

    def zbody(t, _):
        sbuf[pl.ds(t * 16, 16)] = zeros
        return 0

    jax.lax.fori_loop(0, _RPT * _N // 16, zbody, 0)

    inf_v = jnp.full((16,), jnp.inf, f32)
    ones = jnp.ones((16,), f32)

    for g in range(_NG):
        xo = pxv[pl.ds(ib + g * 16, 16)]
        yo = pyv[pl.ds(ib + g * 16, 16)]

        def dbody(cc, _):
            pcx = pxv[pl.ds(cc * 16, 16)]
            pcy = pyv[pl.ds(cc * 16, 16)]
            for k in range(16):
                dx = xo - pcx[k]
                dy = yo - pcy[k]
                d2buf[pl.ds((g * _N + cc * 16 + k) * 16, 16)] = dx * dx + dy * dy
            return 0

        jax.lax.fori_loop(0, _N // 16, dbody, 0)

        for p in range(_K + 1):
            def pbody(j, carry):
                rm, ri = carry
                d2v = d2buf[pl.ds((g * _N + j) * 16, 16)]
                c = d2v < rm
                return jnp.where(c, d2v, rm), jnp.where(c, j, ri)

            rm, ri = jax.lax.fori_loop(
                0, _N, pbody,
                (inf_v, jnp.zeros((16,), jnp.int32)))
            plsc.store_scatter(d2buf, [(g * _N + ri) * 16 + lane], inf_v)
            if p > 0:
                plsc.store_scatter(sbuf, [(g * 16 + lane) * _N + ri], ones)

    pltpu.sync_copy(sbuf, s_hbm.at[pl.ds(wid * _RPT * _N, _RPT * _N)])


def _t(x):
    n = x.shape[1]
    eye = (jax.lax.broadcasted_iota(jnp.int32, (n, n), 0)
           == jax.lax.broadcasted_iota(jnp.int32, (n, n), 1)).astype(jnp.float32)
    return jax.lax.dot_general(eye, x, (((1,), (1,)), ((), ())),
                               preferred_element_type=jnp.float32)


def _body(pts_ref, s_ref, w1_ref, b1_ref, w2_ref, b2_ref, w3_ref, b3_ref,
          wm1a_ref, wm1b_ref, bm1_ref, wm2_ref, bm2_ref, out_ref):
    f32 = jnp.float32
    px_c = pts_ref[0, :, 0:1]
    py_c = pts_ref[0, :, 1:2]

    S = s_ref[0]
    St = _t(S)
    indeg = jnp.sum(St, axis=1, keepdims=True)
    dis = jax.lax.rsqrt(indeg + f32(_K + 1))
    M = S + St

    def dot(a, b):
        return jax.lax.dot_general(a, b, (((1,), (0,)), ((), ())),
                                   preferred_element_type=f32)

    y = px_c * w1_ref[0:1, :] + py_c * w1_ref[1:2, :]
    z = dis * y
    x = jnp.maximum(dis * dot(M, z) + dis * dis * y + b1_ref[...], 0.0)
    for w_ref, b_ref in ((w2_ref, b2_ref), (w3_ref, b3_ref)):
        y = dot(x, w_ref[...])
        z = dis * y
        x = jnp.maximum(dis * dot(M, z) + dis * dis * y + b_ref[...], 0.0)

    A = dot(x, wm1a_ref[...]) + bm1_ref[...]
    At = _t(A)
    Ct = _t(dot(x, wm1b_ref[...]))
    wm2 = wm2_ref[...]

    acc = jnp.zeros((_N, _N), f32)
    CH = 8
    for c in range(_HID // CH):
        a3 = At[c * CH:(c + 1) * CH, :].reshape(CH, _N, 1)
        c3 = Ct[c * CH:(c + 1) * CH, :].reshape(CH, 1, _N)
        w3 = wm2[c * CH:(c + 1) * CH, :].reshape(CH, 1, 1)
        acc = acc + jnp.sum(jnp.maximum(a3 + c3, 0.0) * w3, axis=0)

    logits = acc + bm2_ref[...]
    prob = 1.0 / (1.0 + jnp.exp(-logits))
    row = jax.lax.broadcasted_iota(jnp.int32, (_N, _N), 0)
    col = jax.lax.broadcasted_iota(jnp.int32, (_N, _N), 1)
    upper = jnp.where(col > row, prob, 0.0)
    out_ref[0] = upper + _t(upper)


def kernel(batch_points, W1, b1, W2, b2, W3, b3, Wm1, bm1, Wm2, bm2):
    pts = batch_points.astype(jnp.float32)
    ptst = jnp.transpose(pts, (0, 2, 1))
    S4 = _knn_sc(ptst).reshape(_B, _N, _N)
    full = lambda shape: pl.BlockSpec(shape, lambda b: (0,) * len(shape))
    grid_spec = pl.GridSpec(
        grid=(_B,),
        in_specs=[
            pl.BlockSpec((1, _N, 2), lambda b: (b, 0, 0)),
            pl.BlockSpec((1, _N, _N), lambda b: (b, 0, 0)),
            full((2, _HID)), full((1, _HID)),
            full((_HID, _HID)), full((1, _HID)),
            full((_HID, _HID)), full((1, _HID)),
            full((_HID, _HID)), full((_HID, _HID)), full((1, _HID)),
            full((_HID, 1)), full((1, 1)),
        ],
        out_specs=pl.BlockSpec((1, _N, _N), lambda b: (b, 0, 0)),
    )
    return pl.pallas_call(
        _body,
        grid_spec=grid_spec,
        out_shape=jax.ShapeDtypeStruct((_B, _N, _N), jnp.float32),
    )(pts, S4, W1, b1.reshape(1, _HID), W2, b2.reshape(1, _HID),
      W3, b3.reshape(1, _HID), Wm1[:_HID], Wm1[_HID:], bm1.reshape(1, _HID),
      Wm2, bm2.reshape(1, 1))


# speedup vs baseline: 2.8942x; 2.8942x over previous
import jax
import jax.numpy as jnp
from jax.experimental import pallas as pl
from jax.experimental.pallas import tpu as pltpu

_B, _N, _K = 4, 512, 8
_HID = 64


def _t(x):
    n = x.shape[1]
    eye = (jax.lax.broadcasted_iota(jnp.int32, (n, n), 0)
           == jax.lax.broadcasted_iota(jnp.int32, (n, n), 1)).astype(jnp.float32)
    return jax.lax.dot_general(eye, x, (((1,), (1,)), ((), ())),
                               preferred_element_type=jnp.float32)


def _body(pts_ref, ptst_ref, w1_ref, b1_ref, w2_ref, b2_ref, w3_ref, b3_ref,
          wm1a_ref, wm1b_ref, bm1_ref, wm2_ref, bm2_ref, out_ref):
    f32 = jnp.float32
    px_c = pts_ref[0, :, 0:1]
    py_c = pts_ref[0, :, 1:2]
    px_r = ptst_ref[0, 0:1, :]
    py_r = ptst_ref[0, 1:2, :]

    dx = px_c - px_r
    dy = py_c - py_r
    d2 = dx * dx + dy * dy

    col = jax.lax.broadcasted_iota(jnp.int32, (_N, _N), 1)
    big_idx = jnp.int32(_N)
    inf = f32(jnp.inf)

    S = jnp.zeros((_N, _N), f32)
    D = d2
    for t in range(_K + 1):
        rmin = jnp.min(D, axis=1, keepdims=True)
        cand = jnp.where(D == rmin, col, big_idx)
        first = jnp.min(cand, axis=1, keepdims=True)
        onehot = col == first
        if t > 0:
            S = S + onehot.astype(f32)
        D = jnp.where(onehot, inf, D)

    St = _t(S)
    indeg = jnp.sum(St, axis=1, keepdims=True)
    dis = jax.lax.rsqrt(indeg + f32(_K + 1))
    M = S + St

    def dot(a, b):
        return jax.lax.dot_general(a, b, (((1,), (0,)), ((), ())),
                                   preferred_element_type=f32)

    y = px_c * w1_ref[0:1, :] + py_c * w1_ref[1:2, :]
    z = dis * y
    x = jnp.maximum(dis * dot(M, z) + dis * dis * y + b1_ref[...], 0.0)
    for w_ref, b_ref in ((w2_ref, b2_ref), (w3_ref, b3_ref)):
        y = dot(x, w_ref[...])
        z = dis * y
        x = jnp.maximum(dis * dot(M, z) + dis * dis * y + b_ref[...], 0.0)

    A = dot(x, wm1a_ref[...]) + bm1_ref[...]
    At = _t(A)
    Ct = _t(dot(x, wm1b_ref[...]))
    wm2 = wm2_ref[...]

    CH = 8
    RB = 128
    for bi in range(_N // RB):
        r0 = bi * RB
        W = _N - r0
        acc = jnp.zeros((RB, W), f32)
        for c in range(_HID // CH):
            a3 = At[c * CH:(c + 1) * CH, r0:r0 + RB].reshape(CH, RB, 1)
            c3 = Ct[c * CH:(c + 1) * CH, r0:].reshape(CH, 1, W)
            w3 = wm2[c * CH:(c + 1) * CH, :].reshape(CH, 1, 1)
            acc = acc + jnp.sum(jnp.maximum(a3 + c3, 0.0) * w3, axis=0)
        prob = 1.0 / (1.0 + jnp.exp(-(acc + bm2_ref[...])))
        rl = jax.lax.broadcasted_iota(jnp.int32, (RB, W), 0)
        cl = jax.lax.broadcasted_iota(jnp.int32, (RB, W), 1)
        if r0 > 0:
            out_ref[0, r0:r0 + RB, 0:r0] = jnp.zeros((RB, r0), f32)
        out_ref[0, r0:r0 + RB, r0:] = jnp.where(cl > rl, prob, 0.0)

    upper = out_ref[0]
    out_ref[0] = upper + _t(upper)


def kernel(batch_points, W1, b1, W2, b2, W3, b3, Wm1, bm1, Wm2, bm2):
    pts = batch_points.astype(jnp.float32)
    ptst = jnp.transpose(pts, (0, 2, 1))
    full = lambda shape: pl.BlockSpec(shape, lambda b: (0,) * len(shape))
    grid_spec = pl.GridSpec(
        grid=(_B,),
        in_specs=[
            pl.BlockSpec((1, _N, 2), lambda b: (b, 0, 0)),
            pl.BlockSpec((1, 2, _N), lambda b: (b, 0, 0)),
            full((2, _HID)), full((1, _HID)),
            full((_HID, _HID)), full((1, _HID)),
            full((_HID, _HID)), full((1, _HID)),
            full((_HID, _HID)), full((_HID, _HID)), full((1, _HID)),
            full((_HID, 1)), full((1, 1)),
        ],
        out_specs=pl.BlockSpec((1, _N, _N), lambda b: (b, 0, 0)),
    )
    return pl.pallas_call(
        _body,
        grid_spec=grid_spec,
        out_shape=jax.ShapeDtypeStruct((_B, _N, _N), jnp.float32),
    )(pts, ptst, W1, b1.reshape(1, _HID), W2, b2.reshape(1, _HID),
      W3, b3.reshape(1, _HID), Wm1[:_HID], Wm1[_HID:], bm1.reshape(1, _HID),
      Wm2, bm2.reshape(1, 1))


# speedup vs baseline: 3.0008x; 1.0368x over previous
import jax
import jax.numpy as jnp
from jax.experimental import pallas as pl
from jax.experimental.pallas import tpu as pltpu

_B, _N, _K = 4, 512, 8
_HID = 64


def _t(x):
    n = x.shape[1]
    eye = (jax.lax.broadcasted_iota(jnp.int32, (n, n), 0)
           == jax.lax.broadcasted_iota(jnp.int32, (n, n), 1)).astype(jnp.float32)
    return jax.lax.dot_general(eye, x, (((1,), (1,)), ((), ())),
                               preferred_element_type=jnp.float32)


def _body(pts_ref, ptst_ref, w1_ref, b1_ref, w2_ref, b2_ref, w3_ref, b3_ref,
          wm1a_ref, wm1b_ref, bm1_ref, wm2_ref, bm2_ref, out_ref):
    f32 = jnp.float32
    px_c = pts_ref[0, :, 0:1]
    py_c = pts_ref[0, :, 1:2]
    px_r = ptst_ref[0, 0:1, :]
    py_r = ptst_ref[0, 1:2, :]

    dx = px_c - px_r
    dy = py_c - py_r
    d2 = dx * dx + dy * dy

    col = jax.lax.broadcasted_iota(jnp.int32, (_N, _N), 1)
    big_idx = jnp.int32(_N)
    inf = f32(jnp.inf)

    D = d2
    oh0 = None
    for t in range(_K + 1):
        rmin = jnp.min(D, axis=1, keepdims=True)
        cand = jnp.where(D == rmin, col, big_idx)
        first = jnp.min(cand, axis=1, keepdims=True)
        onehot = col == first
        if t == 0:
            oh0 = onehot
        D = jnp.where(onehot, inf, D)
    S = jnp.where(jnp.logical_and(D == inf, jnp.logical_not(oh0)), 1.0, 0.0)

    St = _t(S)
    indeg = jnp.sum(St, axis=1, keepdims=True)
    dis = jax.lax.rsqrt(indeg + f32(_K + 1))
    M = S + St

    def dot(a, b):
        return jax.lax.dot_general(a, b, (((1,), (0,)), ((), ())),
                                   preferred_element_type=f32)

    y = px_c * w1_ref[0:1, :] + py_c * w1_ref[1:2, :]
    z = dis * y
    x = jnp.maximum(dis * dot(M, z) + dis * dis * y + b1_ref[...], 0.0)
    for w_ref, b_ref in ((w2_ref, b2_ref), (w3_ref, b3_ref)):
        y = dot(x, w_ref[...])
        z = dis * y
        x = jnp.maximum(dis * dot(M, z) + dis * dis * y + b_ref[...], 0.0)

    A = dot(x, wm1a_ref[...]) + bm1_ref[...]
    At = _t(A)
    Ct = _t(dot(x, wm1b_ref[...]))
    wm2 = wm2_ref[...]

    acc = jnp.zeros((_N, _N), f32)
    CH = 8
    for c in range(_HID // CH):
        a3 = At[c * CH:(c + 1) * CH, :].reshape(CH, _N, 1)
        c3 = Ct[c * CH:(c + 1) * CH, :].reshape(CH, 1, _N)
        w3 = wm2[c * CH:(c + 1) * CH, :].reshape(CH, 1, 1)
        acc = acc + jnp.sum(jnp.maximum(a3 + c3, 0.0) * w3, axis=0)

    logits = acc + bm2_ref[...]
    prob = 1.0 / (1.0 + jnp.exp(-logits))
    row = jax.lax.broadcasted_iota(jnp.int32, (_N, _N), 0)
    upper = jnp.where(col > row, prob, 0.0)
    out_ref[0] = upper + _t(upper)


def kernel(batch_points, W1, b1, W2, b2, W3, b3, Wm1, bm1, Wm2, bm2):
    pts = batch_points.astype(jnp.float32)
    ptst = jnp.transpose(pts, (0, 2, 1))
    full = lambda shape: pl.BlockSpec(shape, lambda b: (0,) * len(shape))
    grid_spec = pl.GridSpec(
        grid=(_B,),
        in_specs=[
            pl.BlockSpec((1, _N, 2), lambda b: (b, 0, 0)),
            pl.BlockSpec((1, 2, _N), lambda b: (b, 0, 0)),
            full((2, _HID)), full((1, _HID)),
            full((_HID, _HID)), full((1, _HID)),
            full((_HID, _HID)), full((1, _HID)),
            full((_HID, _HID)), full((_HID, _HID)), full((1, _HID)),
            full((_HID, 1)), full((1, 1)),
        ],
        out_specs=pl.BlockSpec((1, _N, _N), lambda b: (b, 0, 0)),
    )
    return pl.pallas_call(
        _body,
        grid_spec=grid_spec,
        out_shape=jax.ShapeDtypeStruct((_B, _N, _N), jnp.float32),
    )(pts, ptst, W1, b1.reshape(1, _HID), W2, b2.reshape(1, _HID),
      W3, b3.reshape(1, _HID), Wm1[:_HID], Wm1[_HID:], bm1.reshape(1, _HID),
      Wm2, bm2.reshape(1, 1))
